# baseline (speedup 1.0000x reference)
import functools

import jax
import jax.numpy as jnp
from jax import lax
from jax.experimental import pallas as pl
from jax.experimental.pallas import tpu as pltpu
from jax.experimental.pallas import tpu_sc as plsc

IN_NODE = 128
IN_EDGE = 16
OUT_DIM = 16

NW = 32
CHUNK = 1000
SUB = 128


def _node_proj_body(x_ref, wn_ref, psrc_ref, pdst_ref):
    p = jax.lax.dot_general(
        x_ref[...], wn_ref[...], (((1,), (0,)), ((), ())),
        preferred_element_type=jnp.float32)
    psrc_ref[...] = p[:, :OUT_DIM]
    pdst_ref[...] = p[:, OUT_DIM:]


def _node_proj(x, wn):
    n = x.shape[0]
    blk = n // 5
    return pl.pallas_call(
        _node_proj_body,
        grid=(5,),
        in_specs=[
            pl.BlockSpec((blk, IN_NODE), lambda i: (i, 0)),
            pl.BlockSpec((IN_NODE, 2 * OUT_DIM), lambda i: (0, 0)),
        ],
        out_specs=(
            pl.BlockSpec((blk, OUT_DIM), lambda i: (i, 0)),
            pl.BlockSpec((blk, OUT_DIM), lambda i: (i, 0)),
        ),
        out_shape=(
            jax.ShapeDtypeStruct((n, OUT_DIM), jnp.float32),
            jax.ShapeDtypeStruct((n, OUT_DIM), jnp.float32),
        ),
    )(x, wn)


CT = 8
CHUNK_E = CT * 128
FULL_CHUNKS = 9


def _sc_gather_add_body(psrc_hbm, pdst_hbm, ei_hbm,
                        out_hbm, ei_v, rows_v, gt_v,
                        gsem):
    wid = lax.axis_index("s") * 2 + lax.axis_index("c")
    tile0 = 78 * wid + jnp.minimum(wid, 4)
    tail_tiles = 6 + (wid < 4).astype(jnp.int32)
    lane = lax.iota(jnp.int32, 16)

    def do_tiles(tstart, nt):
        pltpu.sync_copy(ei_hbm.at[pl.ds(tstart * 256, nt * 256)],
                        ei_v.at[pl.ds(0, nt * 256)])
        cps = []
        for t in range(nt):
            cps.append(pltpu.async_copy(
                psrc_hbm.at[ei_v.at[pl.ds(256 * t, 128)]],
                rows_v.at[pl.ds(256 * t, 128)], gsem))
            cps.append(pltpu.async_copy(
                pdst_hbm.at[ei_v.at[pl.ds(256 * t + 128, 128)]],
                rows_v.at[pl.ds(256 * t + 128, 128)], gsem))
        for cp in cps:
            cp.wait()

        @plsc.parallel_loop(0, nt * 128, step=8, unroll=4)
        def row_body(i):
            for k in range(8):
                e = i + k
                s_row = 2 * e - (e & 127)
                v = rows_v[s_row] + rows_v[s_row + 128]
                col = jnp.full((16,), e, dtype=jnp.int32)
                plsc.store_scatter(gt_v, [lane, col], v)
        pltpu.sync_copy(gt_v.at[:, pl.ds(0, nt * 128)],
                        out_hbm.at[:, pl.ds(tstart * 128, nt * 128)])

    def chunk_body(k, _):
        do_tiles(tile0 + CT * k, CT)
        return 0
    lax.fori_loop(0, FULL_CHUNKS, chunk_body, 0)

    def tail_body(j, _):
        do_tiles(tile0 + CT * FULL_CHUNKS + j, 1)
        return 0
    lax.fori_loop(0, tail_tiles, tail_body, 0)


def _sc_gather_add(psrc, pdst, ei_flat):
    e = ei_flat.shape[0] // 2
    mesh = plsc.VectorSubcoreMesh(core_axis_name="c", subcore_axis_name="s")
    return pl.kernel(
        _sc_gather_add_body,
        out_type=jax.ShapeDtypeStruct((OUT_DIM, e), jnp.float32),
        mesh=mesh,
        compiler_params=pltpu.CompilerParams(
            use_tc_tiling_on_sc=False, needs_layout_passes=False),
        scratch_types=[
            pltpu.VMEM((2 * CHUNK_E,), jnp.int32),
            pltpu.VMEM((2 * CHUNK_E, OUT_DIM), jnp.float32),
            pltpu.VMEM((OUT_DIM, CHUNK_E), jnp.float32),
            pltpu.SemaphoreType.DMA,
        ],
    )(psrc, pdst, ei_flat)


def _final_body(we_ref, eat_ref, g_ref, out_ref):
    ep = jax.lax.dot_general(
        we_ref[...], eat_ref[...], (((1,), (0,)), ((), ())),
        preferred_element_type=jnp.float32)
    out_ref[...] = ep + g_ref[...]


def _final_tc(we, eat, g):
    e = eat.shape[1]
    blk = 12800
    grid = e // blk
    return pl.pallas_call(
        _final_body,
        grid=(grid,),
        in_specs=[
            pl.BlockSpec((IN_EDGE, IN_EDGE), lambda i: (0, 0)),
            pl.BlockSpec((IN_EDGE, blk), lambda i: (0, i)),
            pl.BlockSpec((OUT_DIM, blk), lambda i: (0, i)),
        ],
        out_specs=pl.BlockSpec((OUT_DIM, blk), lambda i: (0, i)),
        out_shape=jax.ShapeDtypeStruct((OUT_DIM, e), jnp.float32),
    )(we, eat, g)


def kernel(x, edge_index, edge_attr, W):
    x = x.astype(jnp.float32)
    W = W.astype(jnp.float32)
    e = edge_attr.shape[0]
    ei_flat = (edge_index.astype(jnp.int32)
               .reshape(2, e // 128, 128)
               .transpose(1, 0, 2)
               .reshape(-1))

    wn = jnp.concatenate(
        [W[:, :IN_NODE].T, W[:, IN_NODE + IN_EDGE:].T], axis=1)
    psrc, pdst = _node_proj(x, wn)

    g = _sc_gather_add(psrc, pdst, ei_flat)

    we = W[:, IN_NODE:IN_NODE + IN_EDGE]
    eat = edge_attr.T
    out_t = _final_tc(we, eat, g)
    return out_t.T
